# baseline (speedup 1.0000x reference)
import functools

import jax
import jax.numpy as jnp
from jax import lax
from jax.experimental import pallas as pl
from jax.experimental.pallas import tpu as pltpu
from jax.experimental.pallas import tpu_sc as plsc

N_NODES = 10000
N_EDGES = 160000
R = 14
RPAD = 16
MROW = R * RPAD
NA = 9
NP = NA * NA
NPAD = 96

NC = 2
NS = 16
NW = NC * NS
E_PER_W = N_EDGES // NW
E_PER_W_PAD = 5008

EDGE_BLK = 6400


def _table_body(mask_ref, table_ref):
    p_row = lax.broadcasted_iota(jnp.int32, (NPAD, 16), 0)
    a_col = lax.broadcasted_iota(jnp.int32, (NPAD, 16), 1)
    valid = p_row < NP
    i_oh = ((p_row // NA == a_col) & valid).astype(jnp.float32)
    j_oh = ((p_row % NA == a_col) & valid).astype(jnp.float32)
    m16 = mask_ref[...][:16].astype(jnp.float32)
    dn = (((1,), (0,)), ((), ()))
    t_i = lax.dot_general(i_oh, m16, dn, preferred_element_type=jnp.float32)
    t_j = lax.dot_general(j_oh, m16, dn, preferred_element_type=jnp.float32)
    m_row = lax.broadcasted_iota(jnp.int32, (MROW, R), 0)
    r_col = lax.broadcasted_iota(jnp.int32, (MROW, R), 1)
    sel1 = (m_row // RPAD == r_col).astype(jnp.float32)
    sel2 = (m_row % RPAD == r_col).astype(jnp.float32)
    dnt = (((1,), (1,)), ((), ()))
    bra = lax.dot_general(sel1, t_i, dnt, preferred_element_type=jnp.float32)
    ket = lax.dot_general(sel2, t_j, dnt, preferred_element_type=jnp.float32)
    table_ref[...] = (bra * ket).astype(jnp.bfloat16)


def _build_table(out_repid_mask):
    return pl.pallas_call(
        _table_body,
        out_shape=jax.ShapeDtypeStruct((MROW, NPAD), jnp.bfloat16),
    )(out_repid_mask)


def _sc_body(anum_hbm, edge_hbm, pid_out,
             anum_v, s_v, d_v, p_v):
    wid = lax.axis_index("s") * NC + lax.axis_index("c")
    base = wid * E_PER_W

    pltpu.sync_copy(anum_hbm, anum_v)
    pltpu.sync_copy(edge_hbm.at[pl.ds(base, E_PER_W)],
                    s_v.at[pl.ds(0, E_PER_W)])
    pltpu.sync_copy(edge_hbm.at[pl.ds(N_EDGES + base, E_PER_W)],
                    d_v.at[pl.ds(0, E_PER_W)])

    def pid_body(i, carry):
        off = pl.multiple_of(i * 16, 16)
        sv = s_v[pl.ds(off, 16)]
        dv = d_v[pl.ds(off, 16)]
        sv = jnp.minimum(jnp.maximum(sv, 0), N_NODES - 1)
        dv = jnp.minimum(jnp.maximum(dv, 0), N_NODES - 1)
        a_s = plsc.load_gather(anum_v, [sv])
        a_d = plsc.load_gather(anum_v, [dv])
        p_v[pl.ds(off, 16)] = a_s * NA + a_d
        return carry

    lax.fori_loop(0, E_PER_W_PAD // 16, pid_body, 0)
    pltpu.sync_copy(p_v.at[pl.ds(0, E_PER_W)],
                    pid_out.at[pl.ds(base, E_PER_W)])


def _sc_pids(anum, edge_index):
    mesh = plsc.VectorSubcoreMesh(core_axis_name="c", subcore_axis_name="s")
    f = pl.kernel(
        _sc_body,
        out_type=jax.ShapeDtypeStruct((N_EDGES,), jnp.int32),
        mesh=mesh,
        compiler_params=pltpu.CompilerParams(needs_layout_passes=False),
        scratch_types=[
            pltpu.VMEM((N_NODES,), jnp.int32),
            pltpu.VMEM((E_PER_W_PAD,), jnp.int32),
            pltpu.VMEM((E_PER_W_PAD,), jnp.int32),
            pltpu.VMEM((E_PER_W_PAD,), jnp.int32),
        ],
    )
    return f(anum, edge_index.reshape(-1))


N_EBLK = N_EDGES // EDGE_BLK


def _lookup(tab, pid, mult, out_ref):
    blk = pid.shape[-1]
    ioh = lax.broadcasted_iota(jnp.int32, (NPAD, blk), 0)
    oh = (pid * mult == ioh).astype(jnp.bfloat16)
    dn = (((1,), (0,)), ((), ()))
    acc = lax.dot_general(tab, oh, dn, preferred_element_type=jnp.float32)
    val = acc > 0.5
    for r1 in range(R):
        out_ref[r1] = val[r1 * RPAD:r1 * RPAD + R, :]


def _expand_body(pid_ref, anum_ref, table_ref, node_ref, edge_ref):
    i = pl.program_id(0)
    tab = table_ref[...]

    @pl.when(i < N_EBLK)
    def _():
        _lookup(tab, pid_ref[0], 1, edge_ref)

    @pl.when(i == N_EBLK)
    def _():
        _lookup(tab, anum_ref[0], NA + 1, node_ref)


def _expand_all(pid_edge, anum, table):
    g = N_EBLK
    node, edge = pl.pallas_call(
        _expand_body,
        grid=(g + 1,),
        in_specs=[
            pl.BlockSpec((1, 1, EDGE_BLK), lambda i: (jnp.minimum(i, g - 1), 0, 0)),
            pl.BlockSpec((1, 1, N_NODES), lambda i: (0, 0, 0)),
            pl.BlockSpec((MROW, NPAD), lambda i: (0, 0)),
        ],
        out_specs=(
            pl.BlockSpec((R, R, N_NODES), lambda i: (0, 0, 0)),
            pl.BlockSpec((R, R, EDGE_BLK), lambda i: (0, 0, jnp.minimum(i, g - 1))),
        ),
        out_shape=(
            jax.ShapeDtypeStruct((R, R, N_NODES), jnp.bool_),
            jax.ShapeDtypeStruct((R, R, N_EDGES), jnp.bool_),
        ),
        compiler_params=pltpu.CompilerParams(vmem_limit_bytes=100 * 2**20),
    )(pid_edge.reshape(g, 1, EDGE_BLK), anum.reshape(1, 1, N_NODES), table)
    return jnp.transpose(node, (2, 0, 1)), jnp.transpose(edge, (2, 0, 1))


def kernel(atomic_numbers, edge_index, out_repid_mask):
    anum = atomic_numbers.astype(jnp.int32)
    table = _build_table(out_repid_mask)
    pid_edge = _sc_pids(anum, edge_index)
    node_mask, edge_mask = _expand_all(pid_edge, anum, table)
    return (node_mask, edge_mask)


# speedup vs baseline: 1.0288x; 1.0288x over previous
import functools

import jax
import jax.numpy as jnp
from jax import lax
from jax.experimental import pallas as pl
from jax.experimental.pallas import tpu as pltpu
from jax.experimental.pallas import tpu_sc as plsc

N_NODES = 10000
N_EDGES = 160000
R = 14
RPAD = 16
MROW = R * RPAD
NA = 9
NP = NA * NA
NPAD = 96

NC = 2
NS = 16
NW = NC * NS
E_PER_W = N_EDGES // NW
E_PER_W_PAD = 5008

EDGE_BLK = 6400


def _table_body(mask_ref, table_ref):
    p_row = lax.broadcasted_iota(jnp.int32, (NPAD, 16), 0)
    a_col = lax.broadcasted_iota(jnp.int32, (NPAD, 16), 1)
    valid = p_row < NP
    i_oh = ((p_row // NA == a_col) & valid).astype(jnp.float32)
    j_oh = ((p_row % NA == a_col) & valid).astype(jnp.float32)
    m16 = mask_ref[...][:16].astype(jnp.float32)
    dn = (((1,), (0,)), ((), ()))
    t_i = lax.dot_general(i_oh, m16, dn, preferred_element_type=jnp.float32)
    t_j = lax.dot_general(j_oh, m16, dn, preferred_element_type=jnp.float32)
    m_row = lax.broadcasted_iota(jnp.int32, (MROW, R), 0)
    r_col = lax.broadcasted_iota(jnp.int32, (MROW, R), 1)
    sel1 = (m_row // RPAD == r_col).astype(jnp.float32)
    sel2 = (m_row % RPAD == r_col).astype(jnp.float32)
    dnt = (((1,), (1,)), ((), ()))
    bra = lax.dot_general(sel1, t_i, dnt, preferred_element_type=jnp.float32)
    ket = lax.dot_general(sel2, t_j, dnt, preferred_element_type=jnp.float32)
    table_ref[...] = (bra * ket).astype(jnp.bfloat16)


def _build_table(out_repid_mask):
    return pl.pallas_call(
        _table_body,
        out_shape=jax.ShapeDtypeStruct((MROW, NPAD), jnp.bfloat16),
    )(out_repid_mask)


def _sc_body(anum_hbm, edge_hbm, pid_out,
             anum_v, s_v, d_v, p_v):
    wid = lax.axis_index("s") * NC + lax.axis_index("c")
    base = wid * E_PER_W

    pltpu.sync_copy(anum_hbm, anum_v)
    pltpu.sync_copy(edge_hbm.at[pl.ds(base, E_PER_W)],
                    s_v.at[pl.ds(0, E_PER_W)])
    pltpu.sync_copy(edge_hbm.at[pl.ds(N_EDGES + base, E_PER_W)],
                    d_v.at[pl.ds(0, E_PER_W)])

    def pid_body(i, carry):
        off = pl.multiple_of(i * 16, 16)
        sv = s_v[pl.ds(off, 16)]
        dv = d_v[pl.ds(off, 16)]
        sv = jnp.minimum(jnp.maximum(sv, 0), N_NODES - 1)
        dv = jnp.minimum(jnp.maximum(dv, 0), N_NODES - 1)
        a_s = plsc.load_gather(anum_v, [sv])
        a_d = plsc.load_gather(anum_v, [dv])
        p_v[pl.ds(off, 16)] = a_s * NA + a_d
        return carry

    lax.fori_loop(0, E_PER_W_PAD // 16, pid_body, 0)
    pltpu.sync_copy(p_v.at[pl.ds(0, E_PER_W)],
                    pid_out.at[pl.ds(base, E_PER_W)])


def _sc_pids(anum, edge_index):
    mesh = plsc.VectorSubcoreMesh(core_axis_name="c", subcore_axis_name="s")
    f = pl.kernel(
        _sc_body,
        out_type=jax.ShapeDtypeStruct((N_EDGES,), jnp.int32),
        mesh=mesh,
        compiler_params=pltpu.CompilerParams(needs_layout_passes=False),
        scratch_types=[
            pltpu.VMEM((N_NODES,), jnp.int32),
            pltpu.VMEM((E_PER_W_PAD,), jnp.int32),
            pltpu.VMEM((E_PER_W_PAD,), jnp.int32),
            pltpu.VMEM((E_PER_W_PAD,), jnp.int32),
        ],
    )
    return f(anum, edge_index.reshape(-1))


def _expand_body(mult, pid_ref, table_ref, out_ref):
    blk = pid_ref.shape[-1]
    pid = pid_ref[0] * mult
    ioh = lax.broadcasted_iota(jnp.int32, (NPAD, blk), 0)
    oh = (pid == ioh).astype(jnp.bfloat16)
    tab = table_ref[...]
    dn = (((1,), (0,)), ((), ()))
    acc = lax.dot_general(tab, oh, dn, preferred_element_type=jnp.float32)
    val = acc > 0.5
    for r1 in range(R):
        out_ref[r1] = val[r1 * RPAD:r1 * RPAD + R, :]


def _expand(pid, table, blk, mult):
    n = pid.shape[0]
    grid = n // blk
    out = pl.pallas_call(
        functools.partial(_expand_body, mult),
        grid=(grid,),
        in_specs=[
            pl.BlockSpec((1, 1, blk), lambda i: (i, 0, 0)),
            pl.BlockSpec((MROW, NPAD), lambda i: (0, 0)),
        ],
        out_specs=pl.BlockSpec((R, R, blk), lambda i: (0, 0, i)),
        out_shape=jax.ShapeDtypeStruct((R, R, n), jnp.bool_),
        compiler_params=pltpu.CompilerParams(vmem_limit_bytes=100 * 2**20),
    )(pid.reshape(grid, 1, blk), table)
    return jnp.transpose(out, (2, 0, 1))


def kernel(atomic_numbers, edge_index, out_repid_mask):
    anum = atomic_numbers.astype(jnp.int32)
    table = _build_table(out_repid_mask)
    pid_edge = _sc_pids(anum, edge_index)
    edge_mask = _expand(pid_edge, table, EDGE_BLK, 1)
    node_mask = _expand(anum, table, N_NODES, NA + 1)
    return (node_mask, edge_mask)
